# Initial kernel scaffold; baseline (speedup 1.0000x reference)
import jax
import jax.numpy as jnp
from jax.experimental import pallas as pl


def kernel(features, edge_index, W1, b1, W2, b2, conv_w, conv_b):
    raise NotImplementedError("write your pallas kernel here")


# speedup vs baseline: 17.9659x; 17.9659x over previous
import functools

import jax
import jax.numpy as jnp
from jax import lax
from jax.experimental import pallas as pl
from jax.experimental.pallas import tpu as pltpu
from jax.experimental.pallas import tpu_sc as plsc

NC = 2
NS = 16
NW = NC * NS
CHUNK = 128
STAGE = 8
BLOCKS = 49
C_T = STAGE * BLOCKS
E_PAD = NW * C_T * CHUNK


@functools.lru_cache(maxsize=None)
def _make_segsum(n_pad: int, d: int):
    rpt = n_pad // NS
    full = rpt // CHUNK
    tail = rpt - full * CHUNK
    mesh = plsc.VectorSubcoreMesh(
        core_axis_name="c", subcore_axis_name="s",
        num_cores=NC, num_subcores=NS)

    @functools.partial(
        pl.kernel,
        out_type=jax.ShapeDtypeStruct((NC, n_pad, d), jnp.float32),
        mesh=mesh,
        compiler_params=pltpu.CompilerParams(use_tc_tiling_on_sc=False),
        scratch_types=[
            pltpu.VMEM((STAGE, CHUNK), jnp.int32),
            pltpu.VMEM((STAGE, CHUNK), jnp.int32),
            pltpu.VMEM((STAGE, CHUNK, d), jnp.float32),
            pltpu.VMEM_SHARED((n_pad, d), jnp.float32),
            pltpu.SemaphoreType.DMA,
        ],
    )
    def segsum(x_hbm, src_hbm, dst_hbm, zeros_hbm, out_hbm,
               src_v, dst_v, rows_v, acc_sh, sem_g):
        cid = lax.axis_index("c")
        sid = lax.axis_index("s")
        wid = cid * NS + sid
        base = sid * rpt

        pltpu.sync_copy(zeros_hbm, rows_v.at[0])
        for q in range(full):
            pltpu.sync_copy(rows_v.at[0],
                            acc_sh.at[pl.ds(base + q * CHUNK, CHUNK)])
        if tail:
            pltpu.sync_copy(rows_v.at[0, pl.ds(0, tail)],
                            acc_sh.at[pl.ds(base + full * CHUNK, tail)])
        plsc.subcore_barrier()

        def block(b, carry):
            pltpu.sync_copy(src_hbm.at[wid, pl.ds(b * STAGE, STAGE)], src_v)
            pltpu.sync_copy(dst_hbm.at[wid, pl.ds(b * STAGE, STAGE)], dst_v)
            gh = [pltpu.async_copy(x_hbm.at[src_v.at[j]], rows_v.at[j], sem_g)
                  for j in range(STAGE)]
            for h in gh:
                h.wait()
            for j in range(STAGE):
                pltpu.sync_copy(rows_v.at[j], acc_sh.at[dst_v.at[j]], add=True)
            return carry

        lax.fori_loop(0, BLOCKS, block, 0)

        plsc.subcore_barrier()
        for q in range(full):
            pltpu.sync_copy(acc_sh.at[pl.ds(base + q * CHUNK, CHUNK)],
                            rows_v.at[0])
            pltpu.sync_copy(rows_v.at[0],
                            out_hbm.at[cid, pl.ds(base + q * CHUNK, CHUNK)])
        if tail:
            pltpu.sync_copy(acc_sh.at[pl.ds(base + full * CHUNK, tail)],
                            rows_v.at[0, pl.ds(0, tail)])
            pltpu.sync_copy(rows_v.at[0, pl.ds(0, tail)],
                            out_hbm.at[cid, pl.ds(base + full * CHUNK, tail)])

    return segsum


@functools.lru_cache(maxsize=None)
def _make_mm_relu(n_pad: int, din: int, dout: int):
    br = n_pad // 16

    def body(p0, p1, w, b, o):
        agg = p0[...] + p1[...]
        o[...] = jnp.maximum(
            jnp.dot(agg, w[...], preferred_element_type=jnp.float32)
            + b[...], 0.0)

    return pl.pallas_call(
        body,
        grid=(16,),
        in_specs=[
            pl.BlockSpec((br, din), lambda i: (i, 0)),
            pl.BlockSpec((br, din), lambda i: (i, 0)),
            pl.BlockSpec((din, dout), lambda i: (0, 0)),
            pl.BlockSpec((1, dout), lambda i: (0, 0)),
        ],
        out_specs=pl.BlockSpec((br, dout), lambda i: (i, 0)),
        out_shape=jax.ShapeDtypeStruct((n_pad, dout), jnp.float32),
    )


@functools.lru_cache(maxsize=None)
def _make_capsule(n: int):
    def body(h1f, h2f, cwa, cwb, cb, o):
        u = (jnp.dot(cwa[...], h1f[...], preferred_element_type=jnp.float32)
             + jnp.dot(cwb[...], h2f[...], preferred_element_type=jnp.float32)
             + cb[...])
        mag_sq = jnp.sum(u * u, axis=1, keepdims=True)
        mag = jnp.sqrt(mag_sq)
        o[...] = u * (mag / (1.0 + mag_sq))

    return pl.pallas_call(
        body,
        out_shape=jax.ShapeDtypeStruct((2, n), jnp.float32),
    )


def kernel(features, edge_index, W1, b1, W2, b2, conv_w, conv_b):
    n = features.shape[0]
    e = edge_index.shape[1]
    n_pad = ((n + 127) // 128) * 128
    trash = n_pad - n

    pad = E_PAD - e
    pad_ar = jnp.arange(pad, dtype=jnp.int32)
    src_p = jnp.concatenate([edge_index[0], pad_ar % n])
    dst_p = jnp.concatenate([edge_index[1], n + pad_ar % trash])
    src3 = src_p.reshape(NW, C_T, CHUNK)
    dst3 = dst_p.reshape(NW, C_T, CHUNK)

    zeros8 = jnp.zeros((CHUNK, 8), jnp.float32)
    zeros24 = jnp.zeros((CHUNK, 24), jnp.float32)
    feat8 = jnp.pad(features, ((0, 0), (0, 4)))
    W1p = jnp.pad(W1, ((0, 4), (0, 4)))
    b1p = jnp.pad(b1, (0, 4)).reshape(1, 24)
    W2p = jnp.pad(W2, ((0, 4), (0, 0)))

    parts1 = _make_segsum(n_pad, 8)(feat8, src3, dst3, zeros8)
    h1p = _make_mm_relu(n_pad, 8, 24)(parts1[0], parts1[1], W1p, b1p)

    parts2 = _make_segsum(n_pad, 24)(h1p, src3, dst3, zeros24)
    h2 = _make_mm_relu(n_pad, 24, 20)(
        parts2[0], parts2[1], W2p, b2.reshape(1, 20))

    h1f = h1p[:n, :20].reshape(20, n)
    h2f = h2[:n].reshape(20, n)
    s = _make_capsule(n)(h1f, h2f, conv_w[:, 0, :], conv_w[:, 1, :],
                         conv_b.reshape(2, 1))
    return s.reshape(2 * n, 1)


# speedup vs baseline: 20.0653x; 1.1169x over previous
import functools

import jax
import jax.numpy as jnp
from jax import lax
from jax.experimental import pallas as pl
from jax.experimental.pallas import tpu as pltpu
from jax.experimental.pallas import tpu_sc as plsc

NC = 2
NS = 16
NW = NC * NS
CHUNK = 128
STAGE = 6
BLK_E = STAGE * CHUNK


@functools.lru_cache(maxsize=None)
def _make_segsum(n_pad: int, d: int, ept: int):
    rpt = n_pad // NS
    full = rpt // CHUNK
    tail_r = rpt - full * CHUNK
    nblk = ept // BLK_E
    tail_e = ept - nblk * BLK_E
    t_full = tail_e // CHUNK
    t_rem = tail_e - t_full * CHUNK
    mesh = plsc.VectorSubcoreMesh(
        core_axis_name="c", subcore_axis_name="s",
        num_cores=NC, num_subcores=NS)

    @functools.partial(
        pl.kernel,
        out_type=jax.ShapeDtypeStruct((NC, n_pad, d), jnp.float32),
        mesh=mesh,
        compiler_params=pltpu.CompilerParams(use_tc_tiling_on_sc=False),
        scratch_types=[
            pltpu.VMEM((2, BLK_E), jnp.int32),
            pltpu.VMEM((2, BLK_E), jnp.int32),
            pltpu.VMEM((2, STAGE, CHUNK, d), jnp.float32),
            pltpu.VMEM_SHARED((n_pad, d), jnp.float32),
            pltpu.SemaphoreType.DMA,
            pltpu.SemaphoreType.DMA,
        ],
    )
    def segsum(x_hbm, ei_hbm, zeros_hbm, out_hbm,
               src_v, dst_v, rows_v, acc_sh, sem_g, sem_s):
        cid = lax.axis_index("c")
        sid = lax.axis_index("s")
        wid = cid * NS + sid
        ebase = wid * ept
        base = sid * rpt

        pltpu.sync_copy(zeros_hbm, rows_v.at[0, 0])
        for q in range(full):
            pltpu.sync_copy(rows_v.at[0, 0],
                            acc_sh.at[pl.ds(base + q * CHUNK, CHUNK)])
        if tail_r:
            pltpu.sync_copy(rows_v.at[0, 0, pl.ds(0, tail_r)],
                            acc_sh.at[pl.ds(base + full * CHUNK, tail_r)])
        plsc.subcore_barrier()

        def drain_scatters(p, k):
            for j in range(k):
                pltpu.make_async_copy(x_hbm.at[pl.ds(0, CHUNK)],
                                      rows_v.at[p, j], sem_s).wait()

        def run_block(b, p):
            eoff = ebase + b * BLK_E
            pltpu.sync_copy(ei_hbm.at[0, pl.ds(eoff, BLK_E)], src_v.at[p])
            pltpu.sync_copy(ei_hbm.at[1, pl.ds(eoff, BLK_E)], dst_v.at[p])
            gh = [pltpu.async_copy(
                      x_hbm.at[src_v.at[p, pl.ds(j * CHUNK, CHUNK)]],
                      rows_v.at[p, j], sem_g)
                  for j in range(STAGE)]
            for h in gh:
                h.wait()
            for j in range(STAGE):
                pltpu.async_copy(
                    rows_v.at[p, j],
                    acc_sh.at[dst_v.at[p, pl.ds(j * CHUNK, CHUNK)]],
                    sem_s, add=True)

        def block(b, carry):
            p = lax.rem(b, 2)
            @pl.when(b >= 2)
            def _():
                drain_scatters(p, STAGE)
            run_block(b, p)
            return carry

        lax.fori_loop(0, nblk, block, 0, unroll=2)

        for p in range(2):
            @pl.when(nblk >= 2 - p)
            def _(p=p):
                drain_scatters(lax.rem(nblk + p, 2), STAGE)
        if tail_e:
            eoff = ebase + nblk * BLK_E
            pltpu.sync_copy(ei_hbm.at[0, pl.ds(eoff, tail_e)],
                            src_v.at[0, pl.ds(0, tail_e)])
            pltpu.sync_copy(ei_hbm.at[1, pl.ds(eoff, tail_e)],
                            dst_v.at[0, pl.ds(0, tail_e)])
            sizes = [CHUNK] * t_full + ([t_rem] if t_rem else [])
            gh = []
            for j, sz in enumerate(sizes):
                gh.append(pltpu.async_copy(
                    x_hbm.at[src_v.at[0, pl.ds(j * CHUNK, sz)]],
                    rows_v.at[0, j % STAGE, pl.ds(0, sz)], sem_g))
            for h in gh:
                h.wait()
            for j, sz in enumerate(sizes):
                pltpu.sync_copy(
                    rows_v.at[0, j % STAGE, pl.ds(0, sz)],
                    acc_sh.at[dst_v.at[0, pl.ds(j * CHUNK, sz)]], add=True)

        plsc.subcore_barrier()
        for q in range(full):
            pltpu.sync_copy(acc_sh.at[pl.ds(base + q * CHUNK, CHUNK)],
                            rows_v.at[0, 0])
            pltpu.sync_copy(rows_v.at[0, 0],
                            out_hbm.at[cid, pl.ds(base + q * CHUNK, CHUNK)])
        if tail_r:
            pltpu.sync_copy(acc_sh.at[pl.ds(base + full * CHUNK, tail_r)],
                            rows_v.at[0, 0, pl.ds(0, tail_r)])
            pltpu.sync_copy(rows_v.at[0, 0, pl.ds(0, tail_r)],
                            out_hbm.at[cid, pl.ds(base + full * CHUNK, tail_r)])

    return segsum


@functools.lru_cache(maxsize=None)
def _make_mm_relu(n_pad: int, din: int, dout: int):
    br = n_pad // 16

    def body(p0, p1, w, b, o):
        agg = p0[...] + p1[...]
        o[...] = jnp.maximum(
            jnp.dot(agg, w[...], preferred_element_type=jnp.float32)
            + b[...], 0.0)

    return pl.pallas_call(
        body,
        grid=(16,),
        in_specs=[
            pl.BlockSpec((br, din), lambda i: (i, 0)),
            pl.BlockSpec((br, din), lambda i: (i, 0)),
            pl.BlockSpec((din, dout), lambda i: (0, 0)),
            pl.BlockSpec((1, dout), lambda i: (0, 0)),
        ],
        out_specs=pl.BlockSpec((br, dout), lambda i: (i, 0)),
        out_shape=jax.ShapeDtypeStruct((n_pad, dout), jnp.float32),
    )


@functools.lru_cache(maxsize=None)
def _make_capsule(n: int):
    def body(h1f, h2f, cwa, cwb, cb, o):
        u = (jnp.dot(cwa[...], h1f[...], preferred_element_type=jnp.float32)
             + jnp.dot(cwb[...], h2f[...], preferred_element_type=jnp.float32)
             + cb[...])
        mag_sq = jnp.sum(u * u, axis=1, keepdims=True)
        mag = jnp.sqrt(mag_sq)
        o[...] = u * (mag / (1.0 + mag_sq))

    return pl.pallas_call(
        body,
        out_shape=jax.ShapeDtypeStruct((2, n), jnp.float32),
    )


def kernel(features, edge_index, W1, b1, W2, b2, conv_w, conv_b):
    n = features.shape[0]
    e = edge_index.shape[1]
    n_pad = ((n + 127) // 128) * 128
    ept = e // NW

    zeros8 = jnp.zeros((CHUNK, 8), jnp.float32)
    zeros24 = jnp.zeros((CHUNK, 24), jnp.float32)
    feat8 = jnp.pad(features, ((0, 0), (0, 4)))
    W1p = jnp.pad(W1, ((0, 4), (0, 4)))
    b1p = jnp.pad(b1, (0, 4)).reshape(1, 24)
    W2p = jnp.pad(W2, ((0, 4), (0, 0)))

    parts1 = _make_segsum(n_pad, 8, ept)(feat8, edge_index, zeros8)
    h1p = _make_mm_relu(n_pad, 8, 24)(parts1[0], parts1[1], W1p, b1p)

    parts2 = _make_segsum(n_pad, 24, ept)(h1p, edge_index, zeros24)
    h2 = _make_mm_relu(n_pad, 24, 20)(
        parts2[0], parts2[1], W2p, b2.reshape(1, 20))

    h1f = h1p[:n, :20].reshape(20, n)
    h2f = h2[:n].reshape(20, n)
    s = _make_capsule(n)(h1f, h2f, conv_w[:, 0, :], conv_w[:, 1, :],
                         conv_b.reshape(2, 1))
    return s.reshape(2 * n, 1)


# speedup vs baseline: 25.6698x; 1.2793x over previous
import functools

import jax
import jax.numpy as jnp
from jax import lax
from jax.experimental import pallas as pl
from jax.experimental.pallas import tpu as pltpu
from jax.experimental.pallas import tpu_sc as plsc

NC = 2
NS = 16
NW = NC * NS
CHUNK = 128


@functools.lru_cache(maxsize=None)
def _make_segsum(n_pad: int, d: int, ept: int, stage: int):
    blk_e = stage * CHUNK
    rpt = n_pad // NS
    full = rpt // CHUNK
    tail_r = rpt - full * CHUNK
    nblk = ept // blk_e
    tail_e = ept - nblk * blk_e
    t_full = tail_e // CHUNK
    t_rem = tail_e - t_full * CHUNK
    mesh = plsc.VectorSubcoreMesh(
        core_axis_name="c", subcore_axis_name="s",
        num_cores=NC, num_subcores=NS)

    @functools.partial(
        pl.kernel,
        out_type=jax.ShapeDtypeStruct((NC, n_pad, d), jnp.float32),
        mesh=mesh,
        compiler_params=pltpu.CompilerParams(use_tc_tiling_on_sc=False),
        scratch_types=[
            pltpu.VMEM((4, blk_e), jnp.int32),
            pltpu.VMEM((4, blk_e), jnp.int32),
            pltpu.VMEM((2, stage, CHUNK, d), jnp.float32),
            pltpu.VMEM_SHARED((n_pad, d), jnp.float32),
            pltpu.SemaphoreType.DMA,
            pltpu.SemaphoreType.DMA,
            pltpu.SemaphoreType.DMA,
        ],
    )
    def segsum(x_hbm, ei_hbm, zeros_hbm, out_hbm,
               src_v, dst_v, rows_v, acc_sh, sem_g, sem_s, sem_i):
        cid = lax.axis_index("c")
        sid = lax.axis_index("s")
        wid = cid * NS + sid
        ebase = wid * ept
        base = sid * rpt

        pltpu.sync_copy(zeros_hbm, rows_v.at[0, 0])
        for q in range(full):
            pltpu.sync_copy(rows_v.at[0, 0],
                            acc_sh.at[pl.ds(base + q * CHUNK, CHUNK)])
        if tail_r:
            pltpu.sync_copy(rows_v.at[0, 0, pl.ds(0, tail_r)],
                            acc_sh.at[pl.ds(base + full * CHUNK, tail_r)])
        plsc.subcore_barrier()

        def fire_idx(b, slot):
            eoff = ebase + b * blk_e
            pltpu.async_copy(ei_hbm.at[0, pl.ds(eoff, blk_e)],
                             src_v.at[slot], sem_i)
            pltpu.async_copy(ei_hbm.at[1, pl.ds(eoff, blk_e)],
                             dst_v.at[slot], sem_i)

        def wait_idx(slot):
            for ref in (src_v, dst_v):
                pltpu.make_async_copy(ei_hbm.at[0, pl.ds(0, blk_e)],
                                      ref.at[slot], sem_i).wait()

        def drain_scatters(p, k):
            for j in range(k):
                pltpu.make_async_copy(x_hbm.at[pl.ds(0, CHUNK)],
                                      rows_v.at[p, j], sem_s).wait()

        for b in range(min(2, nblk)):
            fire_idx(b, b)

        def block(b, carry):
            p = lax.rem(b, 4)
            wait_idx(p)
            @pl.when(b >= 2)
            def _():
                drain_scatters(lax.rem(b, 2), stage)
            @pl.when(b + 2 < nblk)
            def _():
                fire_idx(b + 2, lax.rem(b + 2, 4))
            gh = [pltpu.async_copy(
                      x_hbm.at[src_v.at[p, pl.ds(j * CHUNK, CHUNK)]],
                      rows_v.at[lax.rem(b, 2), j], sem_g)
                  for j in range(stage)]
            for h in gh:
                h.wait()
            for j in range(stage):
                pltpu.async_copy(
                    rows_v.at[lax.rem(b, 2), j],
                    acc_sh.at[dst_v.at[p, pl.ds(j * CHUNK, CHUNK)]],
                    sem_s, add=True)
            return carry

        lax.fori_loop(0, nblk, block, 0)

        for q in range(2):
            @pl.when(nblk >= 2 - q)
            def _(q=q):
                drain_scatters(lax.rem(nblk + q, 2), stage)
        if tail_e:
            eoff = ebase + nblk * blk_e
            pltpu.sync_copy(ei_hbm.at[0, pl.ds(eoff, tail_e)],
                            src_v.at[0, pl.ds(0, tail_e)])
            pltpu.sync_copy(ei_hbm.at[1, pl.ds(eoff, tail_e)],
                            dst_v.at[0, pl.ds(0, tail_e)])
            sizes = [CHUNK] * t_full + ([t_rem] if t_rem else [])
            gh = []
            for j, sz in enumerate(sizes):
                gh.append(pltpu.async_copy(
                    x_hbm.at[src_v.at[0, pl.ds(j * CHUNK, sz)]],
                    rows_v.at[j % 2, j // 2, pl.ds(0, sz)], sem_g))
            for h in gh:
                h.wait()
            for j, sz in enumerate(sizes):
                pltpu.sync_copy(
                    rows_v.at[j % 2, j // 2, pl.ds(0, sz)],
                    acc_sh.at[dst_v.at[0, pl.ds(j * CHUNK, sz)]], add=True)

        plsc.subcore_barrier()
        for q in range(full):
            pltpu.sync_copy(acc_sh.at[pl.ds(base + q * CHUNK, CHUNK)],
                            rows_v.at[0, 0])
            pltpu.sync_copy(rows_v.at[0, 0],
                            out_hbm.at[cid, pl.ds(base + q * CHUNK, CHUNK)])
        if tail_r:
            pltpu.sync_copy(acc_sh.at[pl.ds(base + full * CHUNK, tail_r)],
                            rows_v.at[0, 0, pl.ds(0, tail_r)])
            pltpu.sync_copy(rows_v.at[0, 0, pl.ds(0, tail_r)],
                            out_hbm.at[cid, pl.ds(base + full * CHUNK, tail_r)])

    return segsum


@functools.lru_cache(maxsize=None)
def _make_mm_relu(n_pad: int, din: int, dout: int):
    br = n_pad // 16

    def body(p0, p1, w, b, o):
        agg = p0[...] + p1[...]
        o[...] = jnp.maximum(
            jnp.dot(agg, w[...], preferred_element_type=jnp.float32)
            + b[...], 0.0)

    return pl.pallas_call(
        body,
        grid=(16,),
        in_specs=[
            pl.BlockSpec((br, din), lambda i: (i, 0)),
            pl.BlockSpec((br, din), lambda i: (i, 0)),
            pl.BlockSpec((din, dout), lambda i: (0, 0)),
            pl.BlockSpec((1, dout), lambda i: (0, 0)),
        ],
        out_specs=pl.BlockSpec((br, dout), lambda i: (i, 0)),
        out_shape=jax.ShapeDtypeStruct((n_pad, dout), jnp.float32),
    )


@functools.lru_cache(maxsize=None)
def _make_capsule(n: int):
    def body(h1f, h2f, cwa, cwb, cb, o):
        u = (jnp.dot(cwa[...], h1f[...], preferred_element_type=jnp.float32)
             + jnp.dot(cwb[...], h2f[...], preferred_element_type=jnp.float32)
             + cb[...])
        mag_sq = jnp.sum(u * u, axis=1, keepdims=True)
        mag = jnp.sqrt(mag_sq)
        o[...] = u * (mag / (1.0 + mag_sq))

    return pl.pallas_call(
        body,
        out_shape=jax.ShapeDtypeStruct((2, n), jnp.float32),
    )


def kernel(features, edge_index, W1, b1, W2, b2, conv_w, conv_b):
    n = features.shape[0]
    e = edge_index.shape[1]
    n_pad = ((n + 127) // 128) * 128
    ept = e // NW

    zeros8 = jnp.zeros((CHUNK, 8), jnp.float32)
    zeros24 = jnp.zeros((CHUNK, 24), jnp.float32)
    feat8 = jnp.pad(features, ((0, 0), (0, 4)))
    W1p = jnp.pad(W1, ((0, 4), (0, 4)))
    b1p = jnp.pad(b1, (0, 4)).reshape(1, 24)
    W2p = jnp.pad(W2, ((0, 4), (0, 0)))

    parts1 = _make_segsum(n_pad, 8, ept, 14)(feat8, edge_index, zeros8)
    h1p = _make_mm_relu(n_pad, 8, 24)(parts1[0], parts1[1], W1p, b1p)

    parts2 = _make_segsum(n_pad, 24, ept, 7)(h1p, edge_index, zeros24)
    h2 = _make_mm_relu(n_pad, 24, 20)(
        parts2[0], parts2[1], W2p, b2.reshape(1, 20))

    h1f = h1p[:n, :20].reshape(20, n)
    h2f = h2[:n].reshape(20, n)
    s = _make_capsule(n)(h1f, h2f, conv_w[:, 0, :], conv_w[:, 1, :],
                         conv_b.reshape(2, 1))
    return s.reshape(2 * n, 1)


# speedup vs baseline: 40.7439x; 1.5872x over previous
import functools

import jax
import jax.numpy as jnp
from jax import lax
from jax.experimental import pallas as pl
from jax.experimental.pallas import tpu as pltpu
from jax.experimental.pallas import tpu_sc as plsc

NC = 2
NS = 16
NW = NC * NS
CHUNK = 128


@functools.lru_cache(maxsize=None)
def _make_segsum(n_pad: int, d: int, ept: int, stage: int):
    blk_e = stage * CHUNK
    rpt = n_pad // NS
    full = rpt // CHUNK
    tail_r = rpt - full * CHUNK
    nblk = ept // blk_e
    tail_e = ept - nblk * blk_e
    t_full = tail_e // CHUNK
    t_rem = tail_e - t_full * CHUNK
    mesh = plsc.VectorSubcoreMesh(
        core_axis_name="c", subcore_axis_name="s",
        num_cores=NC, num_subcores=NS)

    @functools.partial(
        pl.kernel,
        out_type=jax.ShapeDtypeStruct((NC, n_pad, d), jnp.float32),
        mesh=mesh,
        compiler_params=pltpu.CompilerParams(use_tc_tiling_on_sc=False),
        scratch_types=[
            pltpu.VMEM((4, blk_e), jnp.int32),
            pltpu.VMEM((4, blk_e), jnp.int32),
            pltpu.VMEM((2, stage, CHUNK, d), jnp.float32),
            pltpu.VMEM_SHARED((n_pad, d), jnp.float32),
            pltpu.SemaphoreType.DMA,
            pltpu.SemaphoreType.DMA,
            pltpu.SemaphoreType.DMA,
        ],
    )
    def segsum(x_hbm, ei_hbm, zeros_hbm, out_hbm,
               src_v, dst_v, rows_v, acc_sh, sem_g, sem_s, sem_i):
        cid = lax.axis_index("c")
        sid = lax.axis_index("s")
        wid = cid * NS + sid
        ebase = wid * ept
        base = sid * rpt

        pltpu.sync_copy(zeros_hbm, rows_v.at[0, 0])
        for q in range(full):
            pltpu.sync_copy(rows_v.at[0, 0],
                            acc_sh.at[pl.ds(base + q * CHUNK, CHUNK)])
        if tail_r:
            pltpu.sync_copy(rows_v.at[0, 0, pl.ds(0, tail_r)],
                            acc_sh.at[pl.ds(base + full * CHUNK, tail_r)])
        plsc.subcore_barrier()

        def fire_idx(b, slot):
            eoff = ebase + b * blk_e
            pltpu.async_copy(ei_hbm.at[0, pl.ds(eoff, blk_e)],
                             src_v.at[slot], sem_i)
            pltpu.async_copy(ei_hbm.at[1, pl.ds(eoff, blk_e)],
                             dst_v.at[slot], sem_i)

        def wait_idx(slot):
            for ref in (src_v, dst_v):
                pltpu.make_async_copy(ei_hbm.at[0, pl.ds(0, blk_e)],
                                      ref.at[slot], sem_i).wait()

        def drain_scatters(p, k):
            for j in range(k):
                pltpu.make_async_copy(x_hbm.at[pl.ds(0, CHUNK)],
                                      rows_v.at[p, j], sem_s).wait()

        for b in range(min(2, nblk)):
            fire_idx(b, b)

        def block(b, carry):
            p = lax.rem(b, 4)
            wait_idx(p)
            @pl.when(b >= 2)
            def _():
                drain_scatters(lax.rem(b, 2), stage)
            @pl.when(b + 2 < nblk)
            def _():
                fire_idx(b + 2, lax.rem(b + 2, 4))
            gh = [pltpu.async_copy(
                      x_hbm.at[src_v.at[p, pl.ds(j * CHUNK, CHUNK)]],
                      rows_v.at[lax.rem(b, 2), j], sem_g)
                  for j in range(stage)]
            for h in gh:
                h.wait()
            for j in range(stage):
                pltpu.async_copy(
                    rows_v.at[lax.rem(b, 2), j],
                    acc_sh.at[dst_v.at[p, pl.ds(j * CHUNK, CHUNK)]],
                    sem_s, add=True)
            return carry

        lax.fori_loop(0, nblk, block, 0)

        for q in range(2):
            @pl.when(nblk >= 2 - q)
            def _(q=q):
                drain_scatters(lax.rem(nblk + q, 2), stage)
        if tail_e:
            eoff = ebase + nblk * blk_e
            pltpu.sync_copy(ei_hbm.at[0, pl.ds(eoff, tail_e)],
                            src_v.at[0, pl.ds(0, tail_e)])
            pltpu.sync_copy(ei_hbm.at[1, pl.ds(eoff, tail_e)],
                            dst_v.at[0, pl.ds(0, tail_e)])
            sizes = [CHUNK] * t_full + ([t_rem] if t_rem else [])
            gh = []
            for j, sz in enumerate(sizes):
                gh.append(pltpu.async_copy(
                    x_hbm.at[src_v.at[0, pl.ds(j * CHUNK, sz)]],
                    rows_v.at[j % 2, j // 2, pl.ds(0, sz)], sem_g))
            for h in gh:
                h.wait()
            for j, sz in enumerate(sizes):
                pltpu.sync_copy(
                    rows_v.at[j % 2, j // 2, pl.ds(0, sz)],
                    acc_sh.at[dst_v.at[0, pl.ds(j * CHUNK, sz)]], add=True)

        plsc.subcore_barrier()
        for q in range(full):
            pltpu.sync_copy(acc_sh.at[pl.ds(base + q * CHUNK, CHUNK)],
                            rows_v.at[0, 0])
            pltpu.sync_copy(rows_v.at[0, 0],
                            out_hbm.at[cid, pl.ds(base + q * CHUNK, CHUNK)])
        if tail_r:
            pltpu.sync_copy(acc_sh.at[pl.ds(base + full * CHUNK, tail_r)],
                            rows_v.at[0, 0, pl.ds(0, tail_r)])
            pltpu.sync_copy(rows_v.at[0, 0, pl.ds(0, tail_r)],
                            out_hbm.at[cid, pl.ds(base + full * CHUNK, tail_r)])

    return segsum


@functools.lru_cache(maxsize=None)
def _make_mm_packed(rows: int, cin: int, couts: tuple):
    def body(a_ref, *rest):
        k = len(couts)
        m_refs = rest[:k]
        b_refs = rest[k:2 * k]
        o_refs = rest[2 * k:]
        agg = a_ref[0] + a_ref[1]
        for m, b, o in zip(m_refs, b_refs, o_refs):
            o[...] = jnp.maximum(
                jnp.dot(agg, m[...], preferred_element_type=jnp.float32)
                + b[...], 0.0)

    return pl.pallas_call(
        body,
        out_shape=[jax.ShapeDtypeStruct((rows, co), jnp.float32)
                   for co in couts],
    )


def _expand_w(W, din: int, dout: int, cin: int, cout: int):
    import numpy as np
    m = np.arange(cin)
    c = np.arange(cout)
    cond = jnp.asarray(m[:, None] // din == c[None, :] // dout)
    return jnp.where(cond, W[m % din][:, c % dout], 0.0)


@functools.lru_cache(maxsize=None)
def _make_capsule(n: int):
    def body(h1f, h2f, cwa, cwb, cb, o):
        u = (jnp.dot(cwa[...], h1f[...], preferred_element_type=jnp.float32)
             + jnp.dot(cwb[...], h2f[...], preferred_element_type=jnp.float32)
             + cb[...])
        mag_sq = jnp.sum(u * u, axis=1, keepdims=True)
        mag = jnp.sqrt(mag_sq)
        o[...] = u * (mag / (1.0 + mag_sq))

    return pl.pallas_call(
        body,
        out_shape=jax.ShapeDtypeStruct((2, n), jnp.float32),
    )


def kernel(features, edge_index, W1, b1, W2, b2, conv_w, conv_b):
    n = features.shape[0]
    e = edge_index.shape[1]
    n_pad = ((n + 127) // 128) * 128
    ept = e // NW

    zeros8 = jnp.zeros((CHUNK, 8), jnp.float32)
    zeros24 = jnp.zeros((CHUNK, 24), jnp.float32)
    feat8 = jnp.pad(features, ((0, 0), (0, 4)))
    W1p = jnp.pad(W1, ((0, 4), (0, 4)))
    b1p = jnp.pad(b1, (0, 4)).reshape(1, 24)
    W2p = jnp.pad(W2, ((0, 4), (0, 0)))

    rows = n_pad // 32
    b1p24 = jnp.pad(b1, (0, 4))

    parts1 = _make_segsum(n_pad, 8, ept, 14)(feat8, edge_index, zeros8)
    a1 = parts1.reshape(NC, rows, 256)
    m1a = _expand_w(W1p, 8, 24, 256, 768)
    m1b = _expand_w(W1p[:, :20], 8, 20, 256, 640)
    h1sc, h1cap = _make_mm_packed(rows, 256, (768, 640))(
        a1, m1a, m1b, jnp.tile(b1p24, 32).reshape(1, 768),
        jnp.tile(b1, 32).reshape(1, 640))

    parts2 = _make_segsum(n_pad, 24, ept, 7)(
        h1sc.reshape(n_pad, 24), edge_index, zeros24)
    a2 = parts2.reshape(NC, rows, 768)
    m2 = _expand_w(W2p, 24, 20, 768, 640)
    (h2cap,) = _make_mm_packed(rows, 768, (640,))(
        a2, m2, jnp.tile(b2, 32).reshape(1, 640))

    h1f = h1cap.reshape(-1)[:n * 20].reshape(20, n)
    h2f = h2cap.reshape(-1)[:n * 20].reshape(20, n)
    s = _make_capsule(n)(h1f, h2f, conv_w[:, 0, :], conv_w[:, 1, :],
                         conv_b.reshape(2, 1))
    return s.reshape(2 * n, 1)
